# baseline (speedup 1.0000x reference)
import functools

import jax
import jax.numpy as jnp
from jax import lax
from jax.experimental import pallas as pl
from jax.experimental.pallas import tpu as pltpu
from jax.experimental.pallas import tpu_sc as plsc

N = 10000
E = 320000
D = 128
ED = 16

NC = 2
NS = 16
NW = NC * NS

CH = 80
KB = 5
SPLITS = (63, 62)
NSP = len(SPLITS)

ROWS_TILE = 632
NPAD = NS * ROWS_TILE

BE = 2560
BN = 2000


def _full_spec(shape):
    return pl.BlockSpec(shape, lambda i: tuple(0 for _ in shape))


def _dot(a, w):
    return jnp.dot(a.astype(jnp.bfloat16), w.astype(jnp.bfloat16),
                   preferred_element_type=jnp.float32)


def _silu(x):
    return x * jax.nn.sigmoid(x)


def _ln(h, g, bn):
    mu = jnp.mean(h, axis=-1, keepdims=True)
    var = jnp.var(h, axis=-1, keepdims=True)
    return (h - mu) * lax.rsqrt(var + 1e-5) * g + bn


def _project(x_dst, x_src, W1a, W1b, b1):
    nb = N // BN

    def body(xd_ref, xs_ref, wa_ref, wb_ref, b1_ref, out_ref):
        out_ref[0] = (
            _dot(xd_ref[...], wa_ref[...])
            + b1_ref[...]
        )
        out_ref[1] = _dot(xs_ref[...], wb_ref[...])

    return pl.pallas_call(
        body,
        grid=(nb,),
        in_specs=[
            pl.BlockSpec((BN, D), lambda i: (i, 0)),
            pl.BlockSpec((BN, D), lambda i: (i, 0)),
            _full_spec((D, D)),
            _full_spec((D, D)),
            _full_spec((1, D)),
        ],
        out_specs=pl.BlockSpec((2, BN, D), lambda i: (0, i, 0)),
        out_shape=jax.ShapeDtypeStruct((2, N, D), jnp.float32),
    )(x_dst, x_src, W1a, W1b, b1)


def _gather_sum(table, dix, six, nch):
    mesh = plsc.VectorSubcoreMesh(core_axis_name="c", subcore_axis_name="s")
    et = nch * CH
    esp = NW * et
    full = nch // KB
    rem = nch - full * KB

    @functools.partial(
        pl.kernel,
        out_type=jax.ShapeDtypeStruct((esp, D), jnp.float32),
        mesh=mesh,
        scratch_types=[
            pltpu.VMEM((nch, CH), jnp.int32),
            pltpu.VMEM((nch, CH), jnp.int32),
            pltpu.VMEM((2 * KB, CH, D), jnp.float32),
            pltpu.SemaphoreType.DMA,
            pltpu.SemaphoreType.DMA,
        ],
    )
    def k(table_hbm, dix_hbm, six_hbm, out_hbm, dv, sv, bufs, sem, semw):
        cid = lax.axis_index("c")
        sid = lax.axis_index("s")
        wid = sid * NC + cid
        pltpu.sync_copy(dix_hbm.at[wid], dv)
        pltpu.sync_copy(six_hbm.at[wid], sv)

        def chunk_group(jbase, bank, nb):
            gd = [
                pltpu.async_copy(
                    table_hbm.at[dv.at[jbase + b]], bufs.at[bank + b], sem
                )
                for b in range(nb)
            ]
            ga = []
            for b in range(nb):
                gd[b].wait()
                ga.append(
                    pltpu.async_copy(
                        table_hbm.at[sv.at[jbase + b]], bufs.at[bank + b],
                        sem, add=True,
                    )
                )
            for b in range(nb):
                ga[b].wait()
                pltpu.async_copy(
                    bufs.at[bank + b],
                    out_hbm.at[pl.ds(wid * et + (jbase + b) * CH, CH)],
                    semw,
                )

        def drain_writes(n):
            for _ in range(n):
                pltpu.make_async_copy(
                    bufs.at[0], out_hbm.at[pl.ds(0, CH)], semw
                ).wait()

        def body(j, carry):
            bank = (j % 2) * KB

            @pl.when(j >= 2)
            def _():
                drain_writes(KB)

            chunk_group(j * KB, bank, KB)
            return carry

        lax.fori_loop(0, full, body, 0)
        if rem:
            if full >= 2:
                drain_writes(KB)
            chunk_group(full * KB, (full % 2) * KB, rem)
            drain_writes((KB if full >= 1 else 0) + rem)
        else:
            drain_writes(min(full, 2) * KB)

    return k(table, dix, six)


def _scatter_add(e_new, dix, zeros_pad, nch):
    mesh = plsc.VectorSubcoreMesh(core_axis_name="c", subcore_axis_name="s")
    et = nch * CH

    @functools.partial(
        pl.kernel,
        out_type=jax.ShapeDtypeStruct((NC, NPAD, D), jnp.float32),
        mesh=mesh,
        scratch_types=[
            pltpu.VMEM((nch, CH), jnp.int32),
            pltpu.VMEM((3, CH, D), jnp.float32),
            pltpu.VMEM_SHARED((NPAD, D), jnp.float32),
            pltpu.SemaphoreType.DMA,
            pltpu.SemaphoreType.DMA,
        ],
    )
    def k(e_hbm, dix_hbm, z_hbm, out_hbm, dv, bufs, acc, semr, sems):
        cid = lax.axis_index("c")
        sid = lax.axis_index("s")
        wid = sid * NC + cid
        zr0 = sid * ROWS_TILE
        pltpu.sync_copy(z_hbm.at[pl.ds(zr0, ROWS_TILE)], acc.at[pl.ds(zr0, ROWS_TILE)])
        pltpu.sync_copy(dix_hbm.at[wid], dv)
        plsc.subcore_barrier()

        e0 = wid * et
        pltpu.async_copy(e_hbm.at[pl.ds(e0, CH)], bufs.at[0], semr)

        def body(j, carry):
            @pl.when(j >= 2)
            def _():
                pltpu.make_async_copy(bufs.at[0], acc.at[dv.at[0]], sems).wait()

            @pl.when(j + 1 < nch)
            def _():
                pltpu.async_copy(
                    e_hbm.at[pl.ds(e0 + (j + 1) * CH, CH)],
                    bufs.at[(j + 1) % 3], semr,
                )

            pltpu.make_async_copy(e_hbm.at[pl.ds(0, CH)], bufs.at[0], semr).wait()
            pltpu.async_copy(bufs.at[j % 3], acc.at[dv.at[j]], sems, add=True)
            return carry

        lax.fori_loop(0, nch, body, 0)
        for _ in range(2):
            pltpu.make_async_copy(bufs.at[0], acc.at[dv.at[0]], sems).wait()
        plsc.subcore_barrier()
        pltpu.sync_copy(
            acc.at[pl.ds(zr0, ROWS_TILE)], out_hbm.at[cid, pl.ds(zr0, ROWS_TILE)]
        )

    return k(e_new, dix, zeros_pad)


def _edge_mlp(s, ea_or_attr, enc_ws, W1c, W2, W3, b2, b3, g, bn, first_layer):
    esp = s.shape[0]
    nb = esp // BE

    def body(*refs):
        if first_layer:
            (s_ref, a_ref, eW1, eb1, eW2, eb2, eW3, eb3, eg, ebn,
             w1c, w2, w3, b2r, b3r, gr, bnr, out_ref) = refs
            t = _dot(a_ref[...], eW1[...])
            t = _silu(t + eb1[...])
            t = _silu(_dot(t, eW2[...])
                      + eb2[...])
            t = _dot(t, eW3[...]) + eb3[...]
            ea = _ln(t, eg[...], ebn[...])
        else:
            (s_ref, a_ref, w1c, w2, w3, b2r, b3r, gr, bnr, out_ref) = refs
            ea = a_ref[...]
        a = s_ref[...] + _dot(ea, w1c[...])
        h = _silu(a)
        h = _silu(_dot(h, w2[...]) + b2r[...])
        h = _dot(h, w3[...]) + b3r[...]
        out_ref[...] = _ln(h, gr[...], bnr[...]) + ea

    ea_dim = ED if first_layer else D
    in_specs = [
        pl.BlockSpec((BE, D), lambda i: (i, 0)),
        pl.BlockSpec((BE, ea_dim), lambda i: (i, 0)),
    ]
    args = [s, ea_or_attr]
    if first_layer:
        in_specs += [
            _full_spec((ED, D)), _full_spec((1, D)),
            _full_spec((D, D)), _full_spec((1, D)),
            _full_spec((D, D)), _full_spec((1, D)),
            _full_spec((1, D)), _full_spec((1, D)),
        ]
        args += list(enc_ws)
    in_specs += [
        _full_spec((D, D)), _full_spec((D, D)), _full_spec((D, D)),
        _full_spec((1, D)), _full_spec((1, D)),
        _full_spec((1, D)), _full_spec((1, D)),
    ]
    args += [W1c, W2, W3, b2, b3, g, bn]

    return pl.pallas_call(
        body,
        grid=(nb,),
        in_specs=in_specs,
        out_specs=pl.BlockSpec((BE, D), lambda i: (i, 0)),
        out_shape=jax.ShapeDtypeStruct((esp, D), jnp.float32),
    )(*args)


def _node_mlp(x, parts_list, W1, b1, W2, b2, W3, b3, g, bn):
    nb = N // BN

    def body(*refs):
        x_ref = refs[0]
        p_refs = refs[1:1 + NSP]
        (w1_ref, b1r, w2, b2r, w3, b3r, gr, bnr, out_ref) = refs[1 + NSP:]
        xb = x_ref[...]
        agg = sum(p[0] + p[1] for p in p_refs)
        a = (jnp.dot(xb, w1_ref[0:D, :], preferred_element_type=jnp.float32)
             + jnp.dot(agg, w1_ref[D:2 * D, :], preferred_element_type=jnp.float32)
             + b1r[...])
        h = _silu(a)
        h = _silu(_dot(h, w2[...]) + b2r[...])
        h = _dot(h, w3[...]) + b3r[...]
        out_ref[...] = _ln(h, gr[...], bnr[...]) + xb

    part_spec = pl.BlockSpec((NC, BN, D), lambda i: (0, i, 0))
    return pl.pallas_call(
        body,
        grid=(nb,),
        in_specs=[
            pl.BlockSpec((BN, D), lambda i: (i, 0)),
            *([part_spec] * NSP),
            _full_spec((2 * D, D)),
            _full_spec((1, D)),
            _full_spec((D, D)),
            _full_spec((1, D)),
            _full_spec((D, D)),
            _full_spec((1, D)),
            _full_spec((1, D)),
            _full_spec((1, D)),
        ],
        out_specs=pl.BlockSpec((BN, D), lambda i: (i, 0)),
        out_shape=jax.ShapeDtypeStruct((N, D), jnp.float32),
    )(x, *parts_list, W1, b1, W2, b2, W3, b3, g, bn)


def kernel(x_src, x_dst, edge_index, edge_attr,
           enc_W1, enc_b1, enc_W2, enc_b2, enc_W3, enc_b3, enc_g, enc_bn,
           node_W1, node_b1, node_W2, node_b2, node_W3, node_b3, node_g, node_bn,
           edge_W1, edge_b1, edge_W2, edge_b2, edge_W3, edge_b3, edge_g, edge_bn):
    row = lambda v: v.reshape(1, D)
    src = edge_index[0]
    dst = edge_index[1]
    offs, dix, six, attr = [], [], [], []
    o = 0
    for nch in SPLITS:
        esp = NW * nch * CH
        offs.append(o)
        dix.append(dst[o:o + esp].reshape(NW, nch, CH))
        six.append((src[o:o + esp] + N).reshape(NW, nch, CH))
        attr.append(edge_attr[o:o + esp])
        o += esp
    zeros_pad = jnp.zeros((NPAD, D), jnp.float32)
    enc_ws = (enc_W1, row(enc_b1), enc_W2, row(enc_b2), enc_W3, row(enc_b3),
              row(enc_g), row(enc_bn))

    ea = attr
    for i in range(2):
        W1 = edge_W1[i]
        proj = _project(x_dst, x_src, W1[0:D, :], W1[D:2 * D, :],
                        row(edge_b1[i]))
        table = proj.reshape(2 * N, D)
        e_new, parts = [], []
        for h in range(NSP):
            s = _gather_sum(table, dix[h], six[h], SPLITS[h])
            e_new.append(_edge_mlp(
                s, ea[h], enc_ws, W1[2 * D:3 * D, :], edge_W2[i], edge_W3[i],
                row(edge_b2[i]), row(edge_b3[i]), row(edge_g[i]),
                row(edge_bn[i]), first_layer=(i == 0),
            ))
            parts.append(_scatter_add(e_new[h], dix[h], zeros_pad, SPLITS[h]))
        x_dst = _node_mlp(
            x_dst, [p[:, 0:N, :] for p in parts], node_W1[i], row(node_b1[i]),
            node_W2[i], row(node_b2[i]), node_W3[i], row(node_b3[i]),
            row(node_g[i]), row(node_bn[i]),
        )
        ea = e_new
    return x_dst


# speedup vs baseline: 1.0045x; 1.0045x over previous
import functools

import jax
import jax.numpy as jnp
from jax import lax
from jax.experimental import pallas as pl
from jax.experimental.pallas import tpu as pltpu
from jax.experimental.pallas import tpu_sc as plsc

N = 10000
E = 320000
D = 128
ED = 16

NC = 2
NS = 16
NW = NC * NS

CH = 80
KB = 5
SPLITS = (63, 62)
NSP = len(SPLITS)

ROWS_TILE = 632
NPAD = NS * ROWS_TILE

BE = 2560
BN = 2000


def _full_spec(shape):
    return pl.BlockSpec(shape, lambda i: tuple(0 for _ in shape))


def _dot(a, w):
    return jnp.dot(a, w, preferred_element_type=jnp.float32)


def _silu(x):
    return x * jax.nn.sigmoid(x)


def _ln(h, g, bn):
    mu = jnp.mean(h, axis=-1, keepdims=True)
    var = jnp.var(h, axis=-1, keepdims=True)
    return (h - mu) * lax.rsqrt(var + 1e-5) * g + bn


def _project(x_dst, x_src, W1a, W1b, b1):
    nb = N // BN

    def body(xd_ref, xs_ref, wa_ref, wb_ref, b1_ref, out_ref):
        out_ref[0] = (
            _dot(xd_ref[...], wa_ref[...])
            + b1_ref[...]
        )
        out_ref[1] = _dot(xs_ref[...], wb_ref[...])

    return pl.pallas_call(
        body,
        grid=(nb,),
        in_specs=[
            pl.BlockSpec((BN, D), lambda i: (i, 0)),
            pl.BlockSpec((BN, D), lambda i: (i, 0)),
            _full_spec((D, D)),
            _full_spec((D, D)),
            _full_spec((1, D)),
        ],
        out_specs=pl.BlockSpec((2, BN, D), lambda i: (0, i, 0)),
        out_shape=jax.ShapeDtypeStruct((2, N, D), jnp.float32),
    )(x_dst, x_src, W1a, W1b, b1)


def _gather_sum(table, dix, six, nch):
    mesh = plsc.VectorSubcoreMesh(core_axis_name="c", subcore_axis_name="s")
    et = nch * CH
    esp = NW * et
    full = nch // KB
    rem = nch - full * KB

    @functools.partial(
        pl.kernel,
        out_type=jax.ShapeDtypeStruct((esp, D), jnp.float32),
        mesh=mesh,
        scratch_types=[
            pltpu.VMEM((nch, CH), jnp.int32),
            pltpu.VMEM((nch, CH), jnp.int32),
            pltpu.VMEM((2 * KB, CH, D), jnp.float32),
            pltpu.SemaphoreType.DMA,
            pltpu.SemaphoreType.DMA,
        ],
    )
    def k(table_hbm, dix_hbm, six_hbm, out_hbm, dv, sv, bufs, sem, semw):
        cid = lax.axis_index("c")
        sid = lax.axis_index("s")
        wid = sid * NC + cid
        pltpu.sync_copy(dix_hbm.at[wid], dv)
        pltpu.sync_copy(six_hbm.at[wid], sv)

        def chunk_group(jbase, bank, nb):
            gd = [
                pltpu.async_copy(
                    table_hbm.at[dv.at[jbase + b]], bufs.at[bank + b], sem
                )
                for b in range(nb)
            ]
            ga = []
            for b in range(nb):
                gd[b].wait()
                ga.append(
                    pltpu.async_copy(
                        table_hbm.at[sv.at[jbase + b]], bufs.at[bank + b],
                        sem, add=True,
                    )
                )
            for b in range(nb):
                ga[b].wait()
                pltpu.async_copy(
                    bufs.at[bank + b],
                    out_hbm.at[pl.ds(wid * et + (jbase + b) * CH, CH)],
                    semw,
                )

        def drain_writes(n):
            for _ in range(n):
                pltpu.make_async_copy(
                    bufs.at[0], out_hbm.at[pl.ds(0, CH)], semw
                ).wait()

        def body(j, carry):
            bank = (j % 2) * KB

            @pl.when(j >= 2)
            def _():
                drain_writes(KB)

            chunk_group(j * KB, bank, KB)
            return carry

        lax.fori_loop(0, full, body, 0)
        if rem:
            if full >= 2:
                drain_writes(KB)
            chunk_group(full * KB, (full % 2) * KB, rem)
            drain_writes((KB if full >= 1 else 0) + rem)
        else:
            drain_writes(min(full, 2) * KB)

    return k(table, dix, six)


def _scatter_add(e_new, dix, zeros_pad, nch):
    mesh = plsc.VectorSubcoreMesh(core_axis_name="c", subcore_axis_name="s")
    et = nch * CH

    @functools.partial(
        pl.kernel,
        out_type=jax.ShapeDtypeStruct((NC, NPAD, D), jnp.float32),
        mesh=mesh,
        scratch_types=[
            pltpu.VMEM((nch, CH), jnp.int32),
            pltpu.VMEM((3, CH, D), jnp.float32),
            pltpu.VMEM_SHARED((NPAD, D), jnp.float32),
            pltpu.SemaphoreType.DMA,
            pltpu.SemaphoreType.DMA,
        ],
    )
    def k(e_hbm, dix_hbm, z_hbm, out_hbm, dv, bufs, acc, semr, sems):
        cid = lax.axis_index("c")
        sid = lax.axis_index("s")
        wid = sid * NC + cid
        zr0 = sid * ROWS_TILE
        pltpu.sync_copy(z_hbm.at[pl.ds(zr0, ROWS_TILE)], acc.at[pl.ds(zr0, ROWS_TILE)])
        pltpu.sync_copy(dix_hbm.at[wid], dv)
        plsc.subcore_barrier()

        e0 = wid * et
        pltpu.async_copy(e_hbm.at[pl.ds(e0, CH)], bufs.at[0], semr)

        def body(j, carry):
            @pl.when(j >= 2)
            def _():
                pltpu.make_async_copy(bufs.at[0], acc.at[dv.at[0]], sems).wait()

            @pl.when(j + 1 < nch)
            def _():
                pltpu.async_copy(
                    e_hbm.at[pl.ds(e0 + (j + 1) * CH, CH)],
                    bufs.at[(j + 1) % 3], semr,
                )

            pltpu.make_async_copy(e_hbm.at[pl.ds(0, CH)], bufs.at[0], semr).wait()
            pltpu.async_copy(bufs.at[j % 3], acc.at[dv.at[j]], sems, add=True)
            return carry

        lax.fori_loop(0, nch, body, 0)
        for _ in range(2):
            pltpu.make_async_copy(bufs.at[0], acc.at[dv.at[0]], sems).wait()
        plsc.subcore_barrier()
        pltpu.sync_copy(
            acc.at[pl.ds(zr0, ROWS_TILE)], out_hbm.at[cid, pl.ds(zr0, ROWS_TILE)]
        )

    return k(e_new, dix, zeros_pad)


def _edge_mlp(s, ea_or_attr, enc_ws, W1c, W2, W3, b2, b3, g, bn, first_layer):
    esp = s.shape[0]
    nb = esp // BE

    def body(*refs):
        if first_layer:
            (s_ref, a_ref, eW1, eb1, eW2, eb2, eW3, eb3, eg, ebn,
             w1c, w2, w3, b2r, b3r, gr, bnr, out_ref) = refs
            t = _dot(a_ref[...], eW1[...])
            t = _silu(t + eb1[...])
            t = _silu(_dot(t, eW2[...])
                      + eb2[...])
            t = _dot(t, eW3[...]) + eb3[...]
            ea = _ln(t, eg[...], ebn[...])
        else:
            (s_ref, a_ref, w1c, w2, w3, b2r, b3r, gr, bnr, out_ref) = refs
            ea = a_ref[...]
        a = s_ref[...] + _dot(ea, w1c[...])
        h = _silu(a)
        h = _silu(_dot(h, w2[...]) + b2r[...])
        h = _dot(h, w3[...]) + b3r[...]
        out_ref[...] = _ln(h, gr[...], bnr[...]) + ea

    ea_dim = ED if first_layer else D
    in_specs = [
        pl.BlockSpec((BE, D), lambda i: (i, 0)),
        pl.BlockSpec((BE, ea_dim), lambda i: (i, 0)),
    ]
    args = [s, ea_or_attr]
    if first_layer:
        in_specs += [
            _full_spec((ED, D)), _full_spec((1, D)),
            _full_spec((D, D)), _full_spec((1, D)),
            _full_spec((D, D)), _full_spec((1, D)),
            _full_spec((1, D)), _full_spec((1, D)),
        ]
        args += list(enc_ws)
    in_specs += [
        _full_spec((D, D)), _full_spec((D, D)), _full_spec((D, D)),
        _full_spec((1, D)), _full_spec((1, D)),
        _full_spec((1, D)), _full_spec((1, D)),
    ]
    args += [W1c, W2, W3, b2, b3, g, bn]

    return pl.pallas_call(
        body,
        grid=(nb,),
        in_specs=in_specs,
        out_specs=pl.BlockSpec((BE, D), lambda i: (i, 0)),
        out_shape=jax.ShapeDtypeStruct((esp, D), jnp.float32),
    )(*args)


def _node_mlp(x, parts_list, W1, b1, W2, b2, W3, b3, g, bn):
    nb = N // BN

    def body(*refs):
        x_ref = refs[0]
        p_refs = refs[1:1 + NSP]
        (w1_ref, b1r, w2, b2r, w3, b3r, gr, bnr, out_ref) = refs[1 + NSP:]
        xb = x_ref[...]
        agg = sum(p[0] + p[1] for p in p_refs)
        a = (jnp.dot(xb, w1_ref[0:D, :], preferred_element_type=jnp.float32)
             + jnp.dot(agg, w1_ref[D:2 * D, :], preferred_element_type=jnp.float32)
             + b1r[...])
        h = _silu(a)
        h = _silu(_dot(h, w2[...]) + b2r[...])
        h = _dot(h, w3[...]) + b3r[...]
        out_ref[...] = _ln(h, gr[...], bnr[...]) + xb

    part_spec = pl.BlockSpec((NC, BN, D), lambda i: (0, i, 0))
    return pl.pallas_call(
        body,
        grid=(nb,),
        in_specs=[
            pl.BlockSpec((BN, D), lambda i: (i, 0)),
            *([part_spec] * NSP),
            _full_spec((2 * D, D)),
            _full_spec((1, D)),
            _full_spec((D, D)),
            _full_spec((1, D)),
            _full_spec((D, D)),
            _full_spec((1, D)),
            _full_spec((1, D)),
            _full_spec((1, D)),
        ],
        out_specs=pl.BlockSpec((BN, D), lambda i: (i, 0)),
        out_shape=jax.ShapeDtypeStruct((N, D), jnp.float32),
    )(x, *parts_list, W1, b1, W2, b2, W3, b3, g, bn)


def kernel(x_src, x_dst, edge_index, edge_attr,
           enc_W1, enc_b1, enc_W2, enc_b2, enc_W3, enc_b3, enc_g, enc_bn,
           node_W1, node_b1, node_W2, node_b2, node_W3, node_b3, node_g, node_bn,
           edge_W1, edge_b1, edge_W2, edge_b2, edge_W3, edge_b3, edge_g, edge_bn):
    row = lambda v: v.reshape(1, D)
    src = edge_index[0]
    dst = edge_index[1]
    offs, dix, six, attr = [], [], [], []
    o = 0
    for nch in SPLITS:
        esp = NW * nch * CH
        offs.append(o)
        dix.append(dst[o:o + esp].reshape(NW, nch, CH))
        six.append((src[o:o + esp] + N).reshape(NW, nch, CH))
        attr.append(edge_attr[o:o + esp])
        o += esp
    zeros_pad = jnp.zeros((NPAD, D), jnp.float32)
    enc_ws = (enc_W1, row(enc_b1), enc_W2, row(enc_b2), enc_W3, row(enc_b3),
              row(enc_g), row(enc_bn))

    ea = attr
    for i in range(2):
        W1 = edge_W1[i]
        proj = _project(x_dst, x_src, W1[0:D, :], W1[D:2 * D, :],
                        row(edge_b1[i]))
        table = proj.reshape(2 * N, D)
        s = [_gather_sum(table, dix[h], six[h], SPLITS[h]) for h in range(NSP)]
        e_new, parts = [], []
        for h in range(NSP):
            e_new.append(_edge_mlp(
                s[h], ea[h], enc_ws, W1[2 * D:3 * D, :], edge_W2[i], edge_W3[i],
                row(edge_b2[i]), row(edge_b3[i]), row(edge_g[i]),
                row(edge_bn[i]), first_layer=(i == 0),
            ))
            parts.append(_scatter_add(e_new[h], dix[h], zeros_pad, SPLITS[h]))
        x_dst = _node_mlp(
            x_dst, [p[:, 0:N, :] for p in parts], node_W1[i], row(node_b1[i]),
            node_W2[i], row(node_b2[i]), node_W3[i], row(node_b3[i]),
            row(node_g[i]), row(node_bn[i]),
        )
        ea = e_new
    return x_dst


# speedup vs baseline: 1.1266x; 1.1216x over previous
import functools

import jax
import jax.numpy as jnp
from jax import lax
from jax.experimental import pallas as pl
from jax.experimental.pallas import tpu as pltpu
from jax.experimental.pallas import tpu_sc as plsc

N = 10000
E = 320000
D = 128
ED = 16

NC = 2
NS = 16
NW = NC * NS

CH = 80
KB = 5
SPLITS = (63, 62)
NSP = len(SPLITS)

ROWS_TILE = 632
NPAD = NS * ROWS_TILE

BE = 2560
BN = 2000


def _full_spec(shape):
    return pl.BlockSpec(shape, lambda i: tuple(0 for _ in shape))


def _dot(a, w):
    return jnp.dot(a, w, preferred_element_type=jnp.float32)


def _silu(x):
    return x * jax.nn.sigmoid(x)


def _ln(h, g, bn):
    mu = jnp.mean(h, axis=-1, keepdims=True)
    var = jnp.var(h, axis=-1, keepdims=True)
    return (h - mu) * lax.rsqrt(var + 1e-5) * g + bn


def _project(x_dst, x_src, W1a, W1b, b1):
    nb = N // BN

    def body(xd_ref, xs_ref, wa_ref, wb_ref, b1_ref, out_ref):
        out_ref[0] = (
            _dot(xd_ref[...], wa_ref[...])
            + b1_ref[...]
        )
        out_ref[1] = _dot(xs_ref[...], wb_ref[...])

    return pl.pallas_call(
        body,
        grid=(nb,),
        in_specs=[
            pl.BlockSpec((BN, D), lambda i: (i, 0)),
            pl.BlockSpec((BN, D), lambda i: (i, 0)),
            _full_spec((D, D)),
            _full_spec((D, D)),
            _full_spec((1, D)),
        ],
        out_specs=pl.BlockSpec((2, BN, D), lambda i: (0, i, 0)),
        out_shape=jax.ShapeDtypeStruct((2, N, D), jnp.float32),
    )(x_dst, x_src, W1a, W1b, b1)


def _gather_sum(table, dix, six, nch):
    mesh = plsc.VectorSubcoreMesh(core_axis_name="c", subcore_axis_name="s")
    et = nch * CH
    esp = NW * et
    full = nch // KB
    rem = nch - full * KB

    @functools.partial(
        pl.kernel,
        out_type=jax.ShapeDtypeStruct((esp, D), jnp.float32),
        mesh=mesh,
        scratch_types=[
            pltpu.VMEM((nch, CH), jnp.int32),
            pltpu.VMEM((nch, CH), jnp.int32),
            pltpu.VMEM((2 * KB, CH, D), jnp.float32),
            pltpu.SemaphoreType.DMA,
            pltpu.SemaphoreType.DMA,
        ],
    )
    def k(table_hbm, dix_hbm, six_hbm, out_hbm, dv, sv, bufs, sem, semw):
        cid = lax.axis_index("c")
        sid = lax.axis_index("s")
        wid = sid * NC + cid
        pltpu.sync_copy(dix_hbm.at[wid], dv)
        pltpu.sync_copy(six_hbm.at[wid], sv)

        def chunk_group(jbase, bank, nb):
            gd = [
                pltpu.async_copy(
                    table_hbm.at[dv.at[jbase + b]], bufs.at[bank + b], sem
                )
                for b in range(nb)
            ]
            ga = []
            for b in range(nb):
                gd[b].wait()
                ga.append(
                    pltpu.async_copy(
                        table_hbm.at[sv.at[jbase + b]], bufs.at[bank + b],
                        sem, add=True,
                    )
                )
            for b in range(nb):
                ga[b].wait()
                pltpu.async_copy(
                    bufs.at[bank + b],
                    out_hbm.at[pl.ds(wid * et + (jbase + b) * CH, CH)],
                    semw,
                )

        def drain_writes(n):
            for _ in range(n):
                pltpu.make_async_copy(
                    bufs.at[0], out_hbm.at[pl.ds(0, CH)], semw
                ).wait()

        def body(j, carry):
            bank = (j % 2) * KB

            @pl.when(j >= 2)
            def _():
                drain_writes(KB)

            chunk_group(j * KB, bank, KB)
            return carry

        lax.fori_loop(0, full, body, 0)
        if rem:
            if full >= 2:
                drain_writes(KB)
            chunk_group(full * KB, (full % 2) * KB, rem)
            drain_writes((KB if full >= 1 else 0) + rem)
        else:
            drain_writes(min(full, 2) * KB)

    return k(table, dix, six)


def _scatter_add(e_new, dix, zeros_pad, nch):
    mesh = plsc.VectorSubcoreMesh(core_axis_name="c", subcore_axis_name="s")
    et = nch * CH

    @functools.partial(
        pl.kernel,
        out_type=jax.ShapeDtypeStruct((NC, NPAD, D), jnp.float32),
        mesh=mesh,
        scratch_types=[
            pltpu.VMEM((nch, CH), jnp.int32),
            pltpu.VMEM((3, CH, D), jnp.float32),
            pltpu.VMEM_SHARED((NPAD, D), jnp.float32),
            pltpu.SemaphoreType.DMA,
            pltpu.SemaphoreType.DMA,
        ],
    )
    def k(e_hbm, dix_hbm, z_hbm, out_hbm, dv, bufs, acc, semr, sems):
        cid = lax.axis_index("c")
        sid = lax.axis_index("s")
        wid = sid * NC + cid
        zr0 = sid * ROWS_TILE
        pltpu.sync_copy(z_hbm.at[pl.ds(zr0, ROWS_TILE)], acc.at[pl.ds(zr0, ROWS_TILE)])
        pltpu.sync_copy(dix_hbm.at[wid], dv)
        plsc.subcore_barrier()

        e0 = wid * et
        pltpu.async_copy(e_hbm.at[pl.ds(e0, CH)], bufs.at[0], semr)

        def body(j, carry):
            @pl.when(j >= 2)
            def _():
                pltpu.make_async_copy(bufs.at[0], acc.at[dv.at[0]], sems).wait()

            @pl.when(j + 1 < nch)
            def _():
                pltpu.async_copy(
                    e_hbm.at[pl.ds(e0 + (j + 1) * CH, CH)],
                    bufs.at[(j + 1) % 3], semr,
                )

            pltpu.make_async_copy(e_hbm.at[pl.ds(0, CH)], bufs.at[0], semr).wait()
            pltpu.async_copy(bufs.at[j % 3], acc.at[dv.at[j]], sems, add=True)
            return carry

        lax.fori_loop(0, nch, body, 0)
        for _ in range(2):
            pltpu.make_async_copy(bufs.at[0], acc.at[dv.at[0]], sems).wait()
        plsc.subcore_barrier()
        pltpu.sync_copy(
            acc.at[pl.ds(zr0, ROWS_TILE)], out_hbm.at[cid, pl.ds(zr0, ROWS_TILE)]
        )

    return k(e_new, dix, zeros_pad)


def _edge_mlp(s, ea_or_attr, enc_ws, W1c, W2, W3, b2, b3, g, bn, first_layer):
    esp = s.shape[0]
    nb = esp // BE

    def body(*refs):
        if first_layer:
            (s_ref, a_ref, eW1, eb1, eW2, eb2, eW3, eb3, eg, ebn,
             w1c, w2, w3, b2r, b3r, gr, bnr, out_ref) = refs
            t = lax.dot_general(a_ref[...], eW1[...],
                                dimension_numbers=(((0,), (0,)), ((), ())),
                                preferred_element_type=jnp.float32)
            t = _silu(t + eb1[...])
            t = _silu(_dot(t, eW2[...])
                      + eb2[...])
            t = _dot(t, eW3[...]) + eb3[...]
            ea = _ln(t, eg[...], ebn[...])
        else:
            (s_ref, a_ref, w1c, w2, w3, b2r, b3r, gr, bnr, out_ref) = refs
            ea = a_ref[...]
        a = s_ref[...] + _dot(ea, w1c[...])
        h = _silu(a)
        h = _silu(_dot(h, w2[...]) + b2r[...])
        h = _dot(h, w3[...]) + b3r[...]
        out_ref[...] = _ln(h, gr[...], bnr[...]) + ea

    if first_layer:
        off_b = ea_or_attr[1]
        attr_spec = pl.BlockSpec(
            (ED, BE), lambda i, o=off_b: (0, i + o))
        ea_or_attr = ea_or_attr[0]
    else:
        attr_spec = pl.BlockSpec((BE, D), lambda i: (i, 0))
    in_specs = [
        pl.BlockSpec((BE, D), lambda i: (i, 0)),
        attr_spec,
    ]
    args = [s, ea_or_attr]
    if first_layer:
        in_specs += [
            _full_spec((ED, D)), _full_spec((1, D)),
            _full_spec((D, D)), _full_spec((1, D)),
            _full_spec((D, D)), _full_spec((1, D)),
            _full_spec((1, D)), _full_spec((1, D)),
        ]
        args += list(enc_ws)
    in_specs += [
        _full_spec((D, D)), _full_spec((D, D)), _full_spec((D, D)),
        _full_spec((1, D)), _full_spec((1, D)),
        _full_spec((1, D)), _full_spec((1, D)),
    ]
    args += [W1c, W2, W3, b2, b3, g, bn]

    return pl.pallas_call(
        body,
        grid=(nb,),
        in_specs=in_specs,
        out_specs=pl.BlockSpec((BE, D), lambda i: (i, 0)),
        out_shape=jax.ShapeDtypeStruct((esp, D), jnp.float32),
    )(*args)


def _node_mlp(x, parts_list, W1, b1, W2, b2, W3, b3, g, bn):
    nb = N // BN

    def body(*refs):
        x_ref = refs[0]
        p_refs = refs[1:1 + NSP]
        (w1_ref, b1r, w2, b2r, w3, b3r, gr, bnr, out_ref) = refs[1 + NSP:]
        xb = x_ref[...]
        agg = sum(p[0] + p[1] for p in p_refs)
        a = (jnp.dot(xb, w1_ref[0:D, :], preferred_element_type=jnp.float32)
             + jnp.dot(agg, w1_ref[D:2 * D, :], preferred_element_type=jnp.float32)
             + b1r[...])
        h = _silu(a)
        h = _silu(_dot(h, w2[...]) + b2r[...])
        h = _dot(h, w3[...]) + b3r[...]
        out_ref[...] = _ln(h, gr[...], bnr[...]) + xb

    part_spec = pl.BlockSpec((NC, BN, D), lambda i: (0, i, 0))
    return pl.pallas_call(
        body,
        grid=(nb,),
        in_specs=[
            pl.BlockSpec((BN, D), lambda i: (i, 0)),
            *([part_spec] * NSP),
            _full_spec((2 * D, D)),
            _full_spec((1, D)),
            _full_spec((D, D)),
            _full_spec((1, D)),
            _full_spec((D, D)),
            _full_spec((1, D)),
            _full_spec((1, D)),
            _full_spec((1, D)),
        ],
        out_specs=pl.BlockSpec((BN, D), lambda i: (i, 0)),
        out_shape=jax.ShapeDtypeStruct((N, D), jnp.float32),
    )(x, *parts_list, W1, b1, W2, b2, W3, b3, g, bn)


def kernel(x_src, x_dst, edge_index, edge_attr,
           enc_W1, enc_b1, enc_W2, enc_b2, enc_W3, enc_b3, enc_g, enc_bn,
           node_W1, node_b1, node_W2, node_b2, node_W3, node_b3, node_g, node_bn,
           edge_W1, edge_b1, edge_W2, edge_b2, edge_W3, edge_b3, edge_g, edge_bn):
    row = lambda v: v.reshape(1, D)
    src = edge_index[0]
    dst = edge_index[1]
    attr_t = edge_attr.T
    offs, dix, six, attr = [], [], [], []
    o = 0
    for nch in SPLITS:
        esp = NW * nch * CH
        offs.append(o)
        dix.append(dst[o:o + esp].reshape(NW, nch, CH))
        six.append((src[o:o + esp] + N).reshape(NW, nch, CH))
        attr.append((attr_t, o // BE))
        o += esp
    zeros_pad = jnp.zeros((NPAD, D), jnp.float32)
    enc_ws = (enc_W1, row(enc_b1), enc_W2, row(enc_b2), enc_W3, row(enc_b3),
              row(enc_g), row(enc_bn))

    ea = attr
    for i in range(2):
        W1 = edge_W1[i]
        proj = _project(x_dst, x_src, W1[0:D, :], W1[D:2 * D, :],
                        row(edge_b1[i]))
        table = proj.reshape(2 * N, D)
        s = [_gather_sum(table, dix[h], six[h], SPLITS[h]) for h in range(NSP)]
        e_new, parts = [], []
        for h in range(NSP):
            e_new.append(_edge_mlp(
                s[h], ea[h], enc_ws, W1[2 * D:3 * D, :], edge_W2[i], edge_W3[i],
                row(edge_b2[i]), row(edge_b3[i]), row(edge_g[i]),
                row(edge_bn[i]), first_layer=(i == 0),
            ))
            parts.append(_scatter_add(e_new[h], dix[h], zeros_pad, SPLITS[h]))
        x_dst = _node_mlp(
            x_dst, parts, node_W1[i], row(node_b1[i]),
            node_W2[i], row(node_b2[i]), node_W3[i], row(node_b3[i]),
            row(node_g[i]), row(node_bn[i]),
        )
        ea = e_new
    return x_dst
